# baseline (speedup 1.0000x reference)
import functools

import jax
import jax.numpy as jnp
from jax import lax
from jax.experimental import pallas as pl
from jax.experimental.pallas import tpu as pltpu
from jax.experimental.pallas import tpu_sc as plsc

N = 10000
E = 320000
D = 128
V = 100000
NEG_SLOPE = 0.2

NC = 2
NS = 16
L = 16
NW = NC * NS

NPAD = 10240
RPT = NPAD // NW
DE = D + 16
ETOT = E + N
G = 32
NGB = 30
EB = G * NGB
NBLK = 11
EPT = EB * NBLK
EPAD = EPT * NW
RSTRIPE = NPAD // NS

_mesh = plsc.VectorSubcoreMesh(core_axis_name="c", subcore_axis_name="s")


_sc_params = pltpu.CompilerParams(needs_layout_passes=False)


@functools.partial(
    pl.kernel,
    out_type=jax.ShapeDtypeStruct((NPAD, D), jnp.float32),
    mesh=_mesh,
    compiler_params=_sc_params,
    scratch_types=[
        pltpu.VMEM((RPT,), jnp.int32),
        pltpu.VMEM((RPT, D), jnp.float32),
        pltpu.SemaphoreType.DMA,
    ],
)
def _emb_gather(emb_hbm, idx_hbm, out_hbm, idx_v, rows_v, sem):
    wid = lax.axis_index("s") * NC + lax.axis_index("c")
    base = wid * RPT
    pltpu.sync_copy(idx_hbm.at[pl.ds(base, RPT)], idx_v)
    pltpu.async_copy(emb_hbm.at[idx_v], rows_v, sem).wait()
    pltpu.sync_copy(rows_v, out_hbm.at[pl.ds(base, RPT)])


@functools.partial(
    pl.kernel,
    out_type=jax.ShapeDtypeStruct((NC * NPAD, DE), jnp.float32),
    mesh=_mesh,
    compiler_params=pltpu.CompilerParams(
        needs_layout_passes=False, use_tc_tiling_on_sc=False),
    scratch_types=[
        pltpu.VMEM((NGB, G), jnp.int32),
        pltpu.VMEM((NGB, G), jnp.int32),
        pltpu.VMEM((EB,), jnp.float32),
        pltpu.VMEM((NPAD,), jnp.float32),
        pltpu.VMEM((NPAD,), jnp.float32),
        pltpu.VMEM((G, DE), jnp.float32),
        pltpu.VMEM((G, DE), jnp.float32),
        pltpu.VMEM_SHARED((NPAD, DE), jnp.float32),
        pltpu.SemaphoreType.DMA,
        pltpu.SemaphoreType.DMA,
        pltpu.SemaphoreType.DMA,
        pltpu.SemaphoreType.DMA,
    ],
)
def _edge_pass(src_hbm, dst_hbm, ea_hbm, ss_hbm, sd_hbm, xw_hbm, zero_hbm,
               out_hbm, src_v, dst_v, ea_v, ss_v, sd_v, rows0, rows1, u_sh,
               semg0, semg1, sems0, sems1):
    c = lax.axis_index("c")
    s = lax.axis_index("s")
    wid = s * NC + c
    pltpu.sync_copy(ss_hbm, ss_v)
    pltpu.sync_copy(sd_hbm, sd_v)
    rb = s * RSTRIPE
    pltpu.sync_copy(zero_hbm.at[pl.ds(rb, RSTRIPE)],
                    u_sh.at[pl.ds(rb, RSTRIPE)])
    plsc.subcore_barrier()

    bufs = (rows0, rows1)
    semg = (semg0, semg1)
    sems = (sems0, sems1)

    def one_group(g, k):
        buf_a, buf_b = bufs[k], bufs[1 - k]

        @pl.when(g > 0)
        def _drain_prev():
            pltpu.make_async_copy(
                buf_b, u_sh.at[dst_v.at[g]], sems[1 - k]).wait()

        @pl.when(g + 1 < NGB)
        def _prefetch():
            pltpu.async_copy(xw_hbm.at[src_v.at[g + 1]], buf_b, semg[1 - k])

        pltpu.make_async_copy(xw_hbm.at[src_v.at[g]], buf_a, semg[k]).wait()
        for sub in range(G // L):
            sidx = src_v[g, pl.ds(sub * L, L)]
            didx = dst_v[g, pl.ds(sub * L, L)]
            eav = ea_v[pl.ds(g * G + sub * L, L)]
            a = plsc.load_gather(sd_v, [didx]) + plsc.load_gather(ss_v, [sidx])
            a = jnp.where(a >= 0.0, a, NEG_SLOPE * a) * eav
            w = jnp.exp(a)
            for r in range(L):
                row = sub * L + r
                wr = jnp.full((L,), w[r], jnp.float32)
                for cc in range(DE // L):
                    buf_a[row, pl.ds(cc * L, L)] = (
                        buf_a[row, pl.ds(cc * L, L)] * wr)
        pltpu.async_copy(buf_a, u_sh.at[dst_v.at[g]], sems[k], add=True)

    def pair(pp, carry):
        one_group(2 * pp, 0)
        one_group(2 * pp + 1, 1)
        return carry

    def block(b, carry):
        rowb = wid * (EPT // G) + b * NGB
        pltpu.sync_copy(src_hbm.at[pl.ds(rowb, NGB)], src_v)
        pltpu.sync_copy(dst_hbm.at[pl.ds(rowb, NGB)], dst_v)
        pltpu.sync_copy(ea_hbm.at[pl.ds(wid * EPT + b * EB, EB)], ea_v)
        pltpu.async_copy(xw_hbm.at[src_v.at[0]], rows0, semg0)
        lax.fori_loop(0, NGB // 2, pair, carry)
        pltpu.make_async_copy(
            rows1, u_sh.at[dst_v.at[NGB - 1]], sems1).wait()
        return carry

    lax.fori_loop(0, NBLK, block, 0)
    plsc.subcore_barrier()
    pltpu.sync_copy(u_sh.at[pl.ds(rb, RSTRIPE)],
                    out_hbm.at[pl.ds(c * NPAD + rb, RSTRIPE)])


def _ln_matmul(h, g_ref, b_ref, W_ref, A2_ref, xw_ref, ssd_ref):
    m = jnp.mean(h, axis=1, keepdims=True)
    v = jnp.mean((h - m) ** 2, axis=1, keepdims=True)
    xn = (h - m) / jnp.sqrt(v + 1e-5) * g_ref[...] + b_ref[...]
    xw = jnp.dot(xn, W_ref[...], preferred_element_type=jnp.float32)
    xw_ref[:, : D] = xw
    col = lax.broadcasted_iota(jnp.int32, (xw.shape[0], DE - D), 1)
    xw_ref[:, D:] = jnp.where(col == 0, 1.0, 0.0)
    ssd_ref[...] = jnp.dot(xw, A2_ref[...], preferred_element_type=jnp.float32,
                 precision=lax.Precision.HIGHEST)


def _tc_pre_body(h_ref, g_ref, b_ref, W_ref, A2_ref, xw_ref, ssd_ref):
    _ln_matmul(h_ref[...], g_ref, b_ref, W_ref, A2_ref, xw_ref, ssd_ref)


def _tc_mid_body(u0_ref, u1_ref, g_ref, b_ref, W_ref, A2_ref, xw_ref, ssd_ref):
    ssum = u0_ref[...] + u1_ref[...]
    h = jnp.maximum(ssum[:, : D] / (ssum[:, D : D + 1] + 1e-16), 0.0)
    _ln_matmul(h, g_ref, b_ref, W_ref, A2_ref, xw_ref, ssd_ref)


def _tc_fin_body(u0_ref, u1_ref, w1_ref, b1_ref, w2_ref, b2_ref, out_ref,
                 acc_ref):
    i = pl.program_id(0)

    @pl.when(i == 0)
    def _init():
        acc_ref[...] = jnp.zeros_like(acc_ref)

    ssum = u0_ref[...] + u1_ref[...]
    h = jnp.maximum(ssum[:, : D] / (ssum[:, D : D + 1] + 1e-16), 0.0)
    row = i * _BR + lax.broadcasted_iota(jnp.int32, (h.shape[0], 1), 0)
    h = jnp.where(row < N, h, 0.0)
    acc_ref[...] += jnp.sum(h, axis=0, keepdims=True)

    @pl.when(i == pl.num_programs(0) - 1)
    def _done():
        gm = acc_ref[...] * (1.0 / N)
        t = jnp.dot(gm, w1_ref[...], preferred_element_type=jnp.float32)
        t = jnp.maximum(t + b1_ref[...], 0.0)
        out_ref[...] = (
            jnp.dot(t, w2_ref[...], preferred_element_type=jnp.float32)
            + b2_ref[...]
        )


_BR = 512
_NB = NPAD // _BR

_full = lambda shape: pl.BlockSpec(shape, lambda i: (0, 0))
_rows = lambda w: pl.BlockSpec((_BR, w), lambda i: (i, 0))


def _tc_pre(h, g, b, W, A2):
    return pl.pallas_call(
        _tc_pre_body,
        grid=(_NB,),
        in_specs=[_rows(D), _full((1, D)), _full((1, D)), _full((D, D)),
                  _full((D, 2))],
        out_specs=[_rows(DE), _rows(2)],
        out_shape=[jax.ShapeDtypeStruct((NPAD, DE), jnp.float32),
                   jax.ShapeDtypeStruct((NPAD, 2), jnp.float32)],
    )(h, g, b, W, A2)


def _tc_mid(ue, g, b, W, A2):
    u0_spec = pl.BlockSpec((_BR, DE), lambda i: (i, 0))
    u1_spec = pl.BlockSpec((_BR, DE), lambda i: (_NB + i, 0))
    return pl.pallas_call(
        _tc_mid_body,
        grid=(_NB,),
        in_specs=[u0_spec, u1_spec, _full((1, D)), _full((1, D)),
                  _full((D, D)), _full((D, 2))],
        out_specs=[_rows(DE), _rows(2)],
        out_shape=[jax.ShapeDtypeStruct((NPAD, DE), jnp.float32),
                   jax.ShapeDtypeStruct((NPAD, 2), jnp.float32)],
    )(ue, ue, g, b, W, A2)


def _tc_fin(ue, w1, b1, w2, b2):
    u0_spec = pl.BlockSpec((_BR, DE), lambda i: (i, 0))
    u1_spec = pl.BlockSpec((_BR, DE), lambda i: (_NB + i, 0))
    return pl.pallas_call(
        _tc_fin_body,
        grid=(_NB,),
        in_specs=[u0_spec, u1_spec, _full((D, D // 2)), _full((1, D // 2)),
                  _full((D // 2, 2)), _full((1, 2))],
        out_specs=[pl.BlockSpec((1, 2), lambda i: (0, 0))],
        out_shape=[jax.ShapeDtypeStruct((1, 2), jnp.float32)],
        scratch_shapes=[pltpu.VMEM((1, D), jnp.float32)],
    )(ue, ue, w1, b1, w2, b2)[0]


def kernel(x, edge_index, edge_attr, emb, ln0_g, ln0_b, W0, asrc0, adst0,
           ln1_g, ln1_b, W1, asrc1, adst1, mW1, mb1, mW2, mb2):
    f32 = jnp.float32
    x = x.astype(jnp.int32)
    xpad = jnp.concatenate([x, jnp.zeros((NPAD - N,), jnp.int32)])
    ei = edge_index.astype(jnp.int32)
    loops = jnp.arange(N, dtype=jnp.int32)
    npe = EPAD - ETOT
    src = jnp.concatenate([ei[0], loops,
                           jnp.zeros((npe,), jnp.int32)]).reshape(EPAD // G, G)
    dst = jnp.concatenate([ei[1], loops,
                           jnp.full((npe,), N, jnp.int32)]).reshape(EPAD // G, G)
    ea = jnp.concatenate([edge_attr.astype(f32).reshape(-1),
                          jnp.ones((N,), f32), jnp.zeros((npe,), f32)])
    zero_u = jnp.zeros((NPAD, DE), f32)

    h0 = _emb_gather(emb.astype(f32), xpad)

    a20 = jnp.stack([asrc0.astype(f32), adst0.astype(f32)], axis=1)
    xw0, ssd0 = _tc_pre(h0, ln0_g.reshape(1, D), ln0_b.reshape(1, D),
                        W0.astype(f32), a20)
    ue0 = _edge_pass(src, dst, ea, ssd0[:, 0], ssd0[:, 1], xw0, zero_u)

    a21 = jnp.stack([asrc1.astype(f32), adst1.astype(f32)], axis=1)
    xw1, ssd1 = _tc_mid(ue0, ln1_g.reshape(1, D), ln1_b.reshape(1, D),
                        W1.astype(f32), a21)
    ue1 = _edge_pass(src, dst, ea, ssd1[:, 0], ssd1[:, 1], xw1, zero_u)

    return _tc_fin(ue1, mW1.astype(f32), mb1.reshape(1, D // 2),
                   mW2.astype(f32), mb2.reshape(1, 2))


# speedup vs baseline: 1.2428x; 1.2428x over previous
import functools

import jax
import jax.numpy as jnp
from jax import lax
from jax.experimental import pallas as pl
from jax.experimental.pallas import tpu as pltpu
from jax.experimental.pallas import tpu_sc as plsc

N = 10000
E = 320000
D = 128
V = 100000
NEG_SLOPE = 0.2

NC = 2
NS = 16
L = 16
NW = NC * NS

NPAD = 10240
RPT = NPAD // NW
DE = D + 16
ETOT = E + N
G = 32
NGB = 30
EB = G * NGB
NBLK0 = 15
NBLK1 = 7
EPAD = NS * (NBLK0 + NBLK1) * EB
RSTRIPE = NPAD // NS

_mesh = plsc.VectorSubcoreMesh(core_axis_name="c", subcore_axis_name="s")


_sc_params = pltpu.CompilerParams(needs_layout_passes=False)


@functools.partial(
    pl.kernel,
    out_type=jax.ShapeDtypeStruct((NPAD, D), jnp.float32),
    mesh=_mesh,
    compiler_params=_sc_params,
    scratch_types=[
        pltpu.VMEM((RPT,), jnp.int32),
        pltpu.VMEM((RPT, D), jnp.float32),
        pltpu.SemaphoreType.DMA,
    ],
)
def _emb_gather(emb_hbm, idx_hbm, out_hbm, idx_v, rows_v, sem):
    wid = lax.axis_index("s") * NC + lax.axis_index("c")
    base = wid * RPT
    pltpu.sync_copy(idx_hbm.at[pl.ds(base, RPT)], idx_v)
    pltpu.async_copy(emb_hbm.at[idx_v], rows_v, sem).wait()
    pltpu.sync_copy(rows_v, out_hbm.at[pl.ds(base, RPT)])


@functools.partial(
    pl.kernel,
    out_type=jax.ShapeDtypeStruct((NC * NPAD, DE), jnp.float32),
    mesh=_mesh,
    compiler_params=pltpu.CompilerParams(
        needs_layout_passes=False, use_tc_tiling_on_sc=False),
    scratch_types=[
        pltpu.VMEM((NGB, G), jnp.int32),
        pltpu.VMEM((NGB, G), jnp.int32),
        pltpu.VMEM((EB,), jnp.float32),
        pltpu.VMEM((NPAD,), jnp.float32),
        pltpu.VMEM((NPAD,), jnp.float32),
        pltpu.VMEM((G, DE), jnp.float32),
        pltpu.VMEM((G, DE), jnp.float32),
        pltpu.VMEM_SHARED((NPAD, DE), jnp.float32),
        pltpu.SemaphoreType.DMA,
        pltpu.SemaphoreType.DMA,
        pltpu.SemaphoreType.DMA,
        pltpu.SemaphoreType.DMA,
    ],
)
def _edge_pass(src_hbm, dst_hbm, ea_hbm, ss_hbm, sd_hbm, xw_hbm, zero_hbm,
               out_hbm, src_v, dst_v, ea_v, ss_v, sd_v, rows0, rows1, u_sh,
               semg0, semg1, sems0, sems1):
    c = lax.axis_index("c")
    s = lax.axis_index("s")
    nblk = jnp.where(c == 0, NBLK0, NBLK1)
    blk_base = jnp.where(c == 0, s * NBLK0, NS * NBLK0 + s * NBLK1)
    pltpu.sync_copy(ss_hbm, ss_v)
    pltpu.sync_copy(sd_hbm, sd_v)
    rb = s * RSTRIPE
    pltpu.sync_copy(zero_hbm, u_sh.at[pl.ds(rb, RSTRIPE)])
    plsc.subcore_barrier()

    bufs = (rows0, rows1)
    semg = (semg0, semg1)
    sems = (sems0, sems1)

    def one_group(g, k):
        buf_a, buf_b = bufs[k], bufs[1 - k]

        @pl.when(g > 0)
        def _drain_prev():
            pltpu.make_async_copy(
                buf_b, u_sh.at[dst_v.at[g]], sems[1 - k]).wait()

        @pl.when(g + 1 < NGB)
        def _prefetch():
            pltpu.async_copy(xw_hbm.at[src_v.at[g + 1]], buf_b, semg[1 - k])

        pltpu.make_async_copy(xw_hbm.at[src_v.at[g]], buf_a, semg[k]).wait()
        for sub in range(G // L):
            sidx = src_v[g, pl.ds(sub * L, L)]
            didx = dst_v[g, pl.ds(sub * L, L)]
            eav = ea_v[pl.ds(g * G + sub * L, L)]
            a = plsc.load_gather(sd_v, [didx]) + plsc.load_gather(ss_v, [sidx])
            a = jnp.where(a >= 0.0, a, NEG_SLOPE * a) * eav
            w = jnp.exp(a)
            for r in range(L):
                row = sub * L + r
                wr = jnp.full((L,), w[r], jnp.float32)
                for cc in range(DE // L):
                    buf_a[row, pl.ds(cc * L, L)] = (
                        buf_a[row, pl.ds(cc * L, L)] * wr)
        pltpu.async_copy(buf_a, u_sh.at[dst_v.at[g]], sems[k], add=True)

    def pair(pp, carry):
        one_group(2 * pp, 0)
        one_group(2 * pp + 1, 1)
        return carry

    def block(b, carry):
        rowb = (blk_base + b) * NGB
        pltpu.sync_copy(src_hbm.at[pl.ds(rowb, NGB)], src_v)
        pltpu.sync_copy(dst_hbm.at[pl.ds(rowb, NGB)], dst_v)
        pltpu.sync_copy(ea_hbm.at[pl.ds((blk_base + b) * EB, EB)], ea_v)
        pltpu.async_copy(xw_hbm.at[src_v.at[0]], rows0, semg0)
        lax.fori_loop(0, NGB // 2, pair, carry)
        pltpu.make_async_copy(
            rows1, u_sh.at[dst_v.at[NGB - 1]], sems1).wait()
        return carry

    lax.fori_loop(0, nblk, block, 0)
    plsc.subcore_barrier()
    pltpu.sync_copy(u_sh.at[pl.ds(rb, RSTRIPE)],
                    out_hbm.at[pl.ds(c * NPAD + rb, RSTRIPE)])


def _ln_matmul(h, g_ref, b_ref, W_ref, A2_ref, xw_ref, ssd_ref):
    m = jnp.mean(h, axis=1, keepdims=True)
    v = jnp.mean((h - m) ** 2, axis=1, keepdims=True)
    xn = (h - m) / jnp.sqrt(v + 1e-5) * g_ref[...] + b_ref[...]
    xw = jnp.dot(xn, W_ref[...], preferred_element_type=jnp.float32)
    xw_ref[:, : D] = xw
    col = lax.broadcasted_iota(jnp.int32, (xw.shape[0], DE - D), 1)
    xw_ref[:, D:] = jnp.where(col == 0, 1.0, 0.0)
    ssd_ref[...] = jnp.dot(xw, A2_ref[...], preferred_element_type=jnp.float32,
                 precision=lax.Precision.HIGHEST)


def _tc_pre_body(h_ref, g_ref, b_ref, W_ref, A2_ref, xw_ref, ssd_ref):
    _ln_matmul(h_ref[...], g_ref, b_ref, W_ref, A2_ref, xw_ref, ssd_ref)


def _tc_mid_body(u0_ref, u1_ref, g_ref, b_ref, W_ref, A2_ref, xw_ref, ssd_ref):
    ssum = u0_ref[...] + u1_ref[...]
    h = jnp.maximum(ssum[:, : D] / (ssum[:, D : D + 1] + 1e-16), 0.0)
    _ln_matmul(h, g_ref, b_ref, W_ref, A2_ref, xw_ref, ssd_ref)


def _tc_fin_body(u0_ref, u1_ref, w1_ref, b1_ref, w2_ref, b2_ref, out_ref,
                 acc_ref):
    i = pl.program_id(0)

    @pl.when(i == 0)
    def _init():
        acc_ref[...] = jnp.zeros_like(acc_ref)

    ssum = u0_ref[...] + u1_ref[...]
    h = jnp.maximum(ssum[:, : D] / (ssum[:, D : D + 1] + 1e-16), 0.0)
    row = i * _BR + lax.broadcasted_iota(jnp.int32, (h.shape[0], 1), 0)
    h = jnp.where(row < N, h, 0.0)
    acc_ref[...] += jnp.sum(h, axis=0, keepdims=True)

    @pl.when(i == pl.num_programs(0) - 1)
    def _done():
        gm = acc_ref[...] * (1.0 / N)
        t = jnp.dot(gm, w1_ref[...], preferred_element_type=jnp.float32)
        t = jnp.maximum(t + b1_ref[...], 0.0)
        out_ref[...] = (
            jnp.dot(t, w2_ref[...], preferred_element_type=jnp.float32)
            + b2_ref[...]
        )


_BR = 512
_NB = NPAD // _BR

_full = lambda shape: pl.BlockSpec(shape, lambda i: (0, 0))
_rows = lambda w: pl.BlockSpec((_BR, w), lambda i: (i, 0))


def _tc_pre(h, g, b, W, A2):
    return pl.pallas_call(
        _tc_pre_body,
        grid=(_NB,),
        in_specs=[_rows(D), _full((1, D)), _full((1, D)), _full((D, D)),
                  _full((D, 2))],
        out_specs=[_rows(DE), _rows(2)],
        out_shape=[jax.ShapeDtypeStruct((NPAD, DE), jnp.float32),
                   jax.ShapeDtypeStruct((NPAD, 2), jnp.float32)],
    )(h, g, b, W, A2)


def _tc_mid(ue, g, b, W, A2):
    u0_spec = pl.BlockSpec((_BR, DE), lambda i: (i, 0))
    u1_spec = pl.BlockSpec((_BR, DE), lambda i: (_NB + i, 0))
    return pl.pallas_call(
        _tc_mid_body,
        grid=(_NB,),
        in_specs=[u0_spec, u1_spec, _full((1, D)), _full((1, D)),
                  _full((D, D)), _full((D, 2))],
        out_specs=[_rows(DE), _rows(2)],
        out_shape=[jax.ShapeDtypeStruct((NPAD, DE), jnp.float32),
                   jax.ShapeDtypeStruct((NPAD, 2), jnp.float32)],
    )(ue, ue, g, b, W, A2)


def _tc_fin(ue, w1, b1, w2, b2):
    u0_spec = pl.BlockSpec((_BR, DE), lambda i: (i, 0))
    u1_spec = pl.BlockSpec((_BR, DE), lambda i: (_NB + i, 0))
    return pl.pallas_call(
        _tc_fin_body,
        grid=(_NB,),
        in_specs=[u0_spec, u1_spec, _full((D, D // 2)), _full((1, D // 2)),
                  _full((D // 2, 2)), _full((1, 2))],
        out_specs=[pl.BlockSpec((1, 2), lambda i: (0, 0))],
        out_shape=[jax.ShapeDtypeStruct((1, 2), jnp.float32)],
        scratch_shapes=[pltpu.VMEM((1, D), jnp.float32)],
    )(ue, ue, w1, b1, w2, b2)[0]


def kernel(x, edge_index, edge_attr, emb, ln0_g, ln0_b, W0, asrc0, adst0,
           ln1_g, ln1_b, W1, asrc1, adst1, mW1, mb1, mW2, mb2):
    f32 = jnp.float32
    x = x.astype(jnp.int32)
    xpad = jnp.concatenate([x, jnp.zeros((NPAD - N,), jnp.int32)])
    ei = edge_index.astype(jnp.int32)
    loops = jnp.arange(N, dtype=jnp.int32)
    npe = EPAD - ETOT
    src = jnp.concatenate([ei[0], loops,
                           jnp.zeros((npe,), jnp.int32)]).reshape(EPAD // G, G)
    dst = jnp.concatenate([ei[1], loops,
                           jnp.full((npe,), N, jnp.int32)]).reshape(EPAD // G, G)
    ea = jnp.concatenate([edge_attr.astype(f32).reshape(-1),
                          jnp.ones((N,), f32), jnp.zeros((npe,), f32)])
    zero_u = jnp.zeros((RSTRIPE, DE), f32)

    h0 = _emb_gather(emb.astype(f32), xpad)

    a20 = jnp.stack([asrc0.astype(f32), adst0.astype(f32)], axis=1)
    xw0, ssd0 = _tc_pre(h0, ln0_g.reshape(1, D), ln0_b.reshape(1, D),
                        W0.astype(f32), a20)
    ue0 = _edge_pass(src, dst, ea, ssd0[:, 0], ssd0[:, 1], xw0, zero_u)

    a21 = jnp.stack([asrc1.astype(f32), adst1.astype(f32)], axis=1)
    xw1, ssd1 = _tc_mid(ue0, ln1_g.reshape(1, D), ln1_b.reshape(1, D),
                        W1.astype(f32), a21)
    ue1 = _edge_pass(src, dst, ea, ssd1[:, 0], ssd1[:, 1], xw1, zero_u)

    return _tc_fin(ue1, mW1.astype(f32), mb1.reshape(1, D // 2),
                   mW2.astype(f32), mb2.reshape(1, 2))


# speedup vs baseline: 1.2430x; 1.0002x over previous
import functools

import jax
import jax.numpy as jnp
from jax import lax
from jax.experimental import pallas as pl
from jax.experimental.pallas import tpu as pltpu
from jax.experimental.pallas import tpu_sc as plsc

N = 10000
E = 320000
D = 128
V = 100000
NEG_SLOPE = 0.2

NC = 2
NS = 16
L = 16
NW = NC * NS

NPAD = 10240
RPT0 = 448
RPT1 = 192
DE = D + 16
ETOT = E + N
G = 32
NGB = 30
EB = G * NGB
NBLK0 = 15
NBLK1 = 7
EPAD = NS * (NBLK0 + NBLK1) * EB
RSTRIPE = NPAD // NS

_mesh = plsc.VectorSubcoreMesh(core_axis_name="c", subcore_axis_name="s")


_sc_params = pltpu.CompilerParams(needs_layout_passes=False)


@functools.partial(
    pl.kernel,
    out_type=jax.ShapeDtypeStruct((NPAD, D), jnp.float32),
    mesh=_mesh,
    compiler_params=_sc_params,
    scratch_types=[
        pltpu.VMEM((RPT0,), jnp.int32),
        pltpu.VMEM((RPT0, D), jnp.float32),
        pltpu.SemaphoreType.DMA,
    ],
)
def _emb_gather(emb_hbm, idx_hbm, out_hbm, idx_v, rows_v, sem):
    c = lax.axis_index("c")
    s = lax.axis_index("s")

    @pl.when(c == 0)
    def _fast():
        base = s * RPT0
        pltpu.sync_copy(idx_hbm.at[pl.ds(base, RPT0)], idx_v)
        pltpu.async_copy(emb_hbm.at[idx_v], rows_v, sem).wait()
        pltpu.sync_copy(rows_v, out_hbm.at[pl.ds(base, RPT0)])

    @pl.when(c != 0)
    def _slow():
        base = NS * RPT0 + s * RPT1
        idx1 = idx_v.at[pl.ds(0, RPT1)]
        rows1 = rows_v.at[pl.ds(0, RPT1)]
        pltpu.sync_copy(idx_hbm.at[pl.ds(base, RPT1)], idx1)
        pltpu.async_copy(emb_hbm.at[idx1], rows1, sem).wait()
        pltpu.sync_copy(rows1, out_hbm.at[pl.ds(base, RPT1)])


@functools.partial(
    pl.kernel,
    out_type=jax.ShapeDtypeStruct((NC * NPAD, DE), jnp.float32),
    mesh=_mesh,
    compiler_params=pltpu.CompilerParams(
        needs_layout_passes=False, use_tc_tiling_on_sc=False),
    scratch_types=[
        pltpu.VMEM((NGB, G), jnp.int32),
        pltpu.VMEM((NGB, G), jnp.int32),
        pltpu.VMEM((EB,), jnp.float32),
        pltpu.VMEM((NPAD,), jnp.float32),
        pltpu.VMEM((NPAD,), jnp.float32),
        pltpu.VMEM((G, DE), jnp.float32),
        pltpu.VMEM((G, DE), jnp.float32),
        pltpu.VMEM_SHARED((NPAD, DE), jnp.float32),
        pltpu.SemaphoreType.DMA,
        pltpu.SemaphoreType.DMA,
        pltpu.SemaphoreType.DMA,
        pltpu.SemaphoreType.DMA,
    ],
)
def _edge_pass(src_hbm, dst_hbm, ea_hbm, ss_hbm, sd_hbm, xw_hbm, zero_hbm,
               out_hbm, src_v, dst_v, ea_v, ss_v, sd_v, rows0, rows1, u_sh,
               semg0, semg1, sems0, sems1):
    c = lax.axis_index("c")
    s = lax.axis_index("s")
    nblk = jnp.where(c == 0, NBLK0, NBLK1)
    blk_base = jnp.where(c == 0, s * NBLK0, NS * NBLK0 + s * NBLK1)
    pltpu.sync_copy(ss_hbm, ss_v)
    pltpu.sync_copy(sd_hbm, sd_v)
    rb = s * RSTRIPE
    pltpu.sync_copy(zero_hbm, u_sh.at[pl.ds(rb, RSTRIPE)])
    plsc.subcore_barrier()

    bufs = (rows0, rows1)
    semg = (semg0, semg1)
    sems = (sems0, sems1)

    def one_group(g, k):
        buf_a, buf_b = bufs[k], bufs[1 - k]

        @pl.when(g > 0)
        def _drain_prev():
            pltpu.make_async_copy(
                buf_b, u_sh.at[dst_v.at[g]], sems[1 - k]).wait()

        @pl.when(g + 1 < NGB)
        def _prefetch():
            pltpu.async_copy(xw_hbm.at[src_v.at[g + 1]], buf_b, semg[1 - k])

        pltpu.make_async_copy(xw_hbm.at[src_v.at[g]], buf_a, semg[k]).wait()
        for sub in range(G // L):
            sidx = src_v[g, pl.ds(sub * L, L)]
            didx = dst_v[g, pl.ds(sub * L, L)]
            eav = ea_v[pl.ds(g * G + sub * L, L)]
            a = plsc.load_gather(sd_v, [didx]) + plsc.load_gather(ss_v, [sidx])
            a = jnp.where(a >= 0.0, a, NEG_SLOPE * a) * eav
            w = jnp.exp(a)
            for r in range(L):
                row = sub * L + r
                wr = jnp.full((L,), w[r], jnp.float32)
                for cc in range(DE // L):
                    buf_a[row, pl.ds(cc * L, L)] = (
                        buf_a[row, pl.ds(cc * L, L)] * wr)
        pltpu.async_copy(buf_a, u_sh.at[dst_v.at[g]], sems[k], add=True)

    def pair(pp, carry):
        one_group(2 * pp, 0)
        one_group(2 * pp + 1, 1)
        return carry

    def block(b, carry):
        rowb = (blk_base + b) * NGB
        pltpu.sync_copy(src_hbm.at[pl.ds(rowb, NGB)], src_v)
        pltpu.sync_copy(dst_hbm.at[pl.ds(rowb, NGB)], dst_v)
        pltpu.sync_copy(ea_hbm.at[pl.ds((blk_base + b) * EB, EB)], ea_v)
        pltpu.async_copy(xw_hbm.at[src_v.at[0]], rows0, semg0)
        lax.fori_loop(0, NGB // 2, pair, carry)
        pltpu.make_async_copy(
            rows1, u_sh.at[dst_v.at[NGB - 1]], sems1).wait()
        return carry

    lax.fori_loop(0, nblk, block, 0)
    plsc.subcore_barrier()
    pltpu.sync_copy(u_sh.at[pl.ds(rb, RSTRIPE)],
                    out_hbm.at[pl.ds(c * NPAD + rb, RSTRIPE)])


def _ln_matmul(h, g_ref, b_ref, W_ref, A2_ref, xw_ref, ssd_ref):
    m = jnp.mean(h, axis=1, keepdims=True)
    v = jnp.mean((h - m) ** 2, axis=1, keepdims=True)
    xn = (h - m) / jnp.sqrt(v + 1e-5) * g_ref[...] + b_ref[...]
    xw = jnp.dot(xn, W_ref[...], preferred_element_type=jnp.float32)
    xw_ref[:, : D] = xw
    col = lax.broadcasted_iota(jnp.int32, (xw.shape[0], DE - D), 1)
    xw_ref[:, D:] = jnp.where(col == 0, 1.0, 0.0)
    ssd_ref[...] = jnp.dot(xw, A2_ref[...], preferred_element_type=jnp.float32,
                 precision=lax.Precision.HIGHEST)


def _tc_pre_body(h_ref, g_ref, b_ref, W_ref, A2_ref, xw_ref, ssd_ref):
    _ln_matmul(h_ref[...], g_ref, b_ref, W_ref, A2_ref, xw_ref, ssd_ref)


def _tc_mid_body(u0_ref, u1_ref, g_ref, b_ref, W_ref, A2_ref, xw_ref, ssd_ref):
    ssum = u0_ref[...] + u1_ref[...]
    h = jnp.maximum(ssum[:, : D] / (ssum[:, D : D + 1] + 1e-16), 0.0)
    _ln_matmul(h, g_ref, b_ref, W_ref, A2_ref, xw_ref, ssd_ref)


def _tc_fin_body(u0_ref, u1_ref, w1_ref, b1_ref, w2_ref, b2_ref, out_ref,
                 acc_ref):
    i = pl.program_id(0)

    @pl.when(i == 0)
    def _init():
        acc_ref[...] = jnp.zeros_like(acc_ref)

    ssum = u0_ref[...] + u1_ref[...]
    h = jnp.maximum(ssum[:, : D] / (ssum[:, D : D + 1] + 1e-16), 0.0)
    row = i * _BR + lax.broadcasted_iota(jnp.int32, (h.shape[0], 1), 0)
    h = jnp.where(row < N, h, 0.0)
    acc_ref[...] += jnp.sum(h, axis=0, keepdims=True)

    @pl.when(i == pl.num_programs(0) - 1)
    def _done():
        gm = acc_ref[...] * (1.0 / N)
        t = jnp.dot(gm, w1_ref[...], preferred_element_type=jnp.float32)
        t = jnp.maximum(t + b1_ref[...], 0.0)
        out_ref[...] = (
            jnp.dot(t, w2_ref[...], preferred_element_type=jnp.float32)
            + b2_ref[...]
        )


_BR = 512
_NB = NPAD // _BR

_full = lambda shape: pl.BlockSpec(shape, lambda i: (0, 0))
_rows = lambda w: pl.BlockSpec((_BR, w), lambda i: (i, 0))


def _tc_pre(h, g, b, W, A2):
    return pl.pallas_call(
        _tc_pre_body,
        grid=(_NB,),
        in_specs=[_rows(D), _full((1, D)), _full((1, D)), _full((D, D)),
                  _full((D, 2))],
        out_specs=[_rows(DE), _rows(2)],
        out_shape=[jax.ShapeDtypeStruct((NPAD, DE), jnp.float32),
                   jax.ShapeDtypeStruct((NPAD, 2), jnp.float32)],
    )(h, g, b, W, A2)


def _tc_mid(ue, g, b, W, A2):
    u0_spec = pl.BlockSpec((_BR, DE), lambda i: (i, 0))
    u1_spec = pl.BlockSpec((_BR, DE), lambda i: (_NB + i, 0))
    return pl.pallas_call(
        _tc_mid_body,
        grid=(_NB,),
        in_specs=[u0_spec, u1_spec, _full((1, D)), _full((1, D)),
                  _full((D, D)), _full((D, 2))],
        out_specs=[_rows(DE), _rows(2)],
        out_shape=[jax.ShapeDtypeStruct((NPAD, DE), jnp.float32),
                   jax.ShapeDtypeStruct((NPAD, 2), jnp.float32)],
    )(ue, ue, g, b, W, A2)


def _tc_fin(ue, w1, b1, w2, b2):
    u0_spec = pl.BlockSpec((_BR, DE), lambda i: (i, 0))
    u1_spec = pl.BlockSpec((_BR, DE), lambda i: (_NB + i, 0))
    return pl.pallas_call(
        _tc_fin_body,
        grid=(_NB,),
        in_specs=[u0_spec, u1_spec, _full((D, D // 2)), _full((1, D // 2)),
                  _full((D // 2, 2)), _full((1, 2))],
        out_specs=[pl.BlockSpec((1, 2), lambda i: (0, 0))],
        out_shape=[jax.ShapeDtypeStruct((1, 2), jnp.float32)],
        scratch_shapes=[pltpu.VMEM((1, D), jnp.float32)],
    )(ue, ue, w1, b1, w2, b2)[0]


def kernel(x, edge_index, edge_attr, emb, ln0_g, ln0_b, W0, asrc0, adst0,
           ln1_g, ln1_b, W1, asrc1, adst1, mW1, mb1, mW2, mb2):
    f32 = jnp.float32
    x = x.astype(jnp.int32)
    xpad = jnp.concatenate([x, jnp.zeros((NPAD - N,), jnp.int32)])
    ei = edge_index.astype(jnp.int32)
    loops = jnp.arange(N, dtype=jnp.int32)
    npe = EPAD - ETOT
    src = jnp.concatenate([ei[0], loops,
                           jnp.zeros((npe,), jnp.int32)]).reshape(EPAD // G, G)
    dst = jnp.concatenate([ei[1], loops,
                           jnp.full((npe,), N, jnp.int32)]).reshape(EPAD // G, G)
    ea = jnp.concatenate([edge_attr.astype(f32).reshape(-1),
                          jnp.ones((N,), f32), jnp.zeros((npe,), f32)])
    zero_u = jnp.zeros((RSTRIPE, DE), f32)

    h0 = _emb_gather(emb.astype(f32), xpad)

    a20 = jnp.stack([asrc0.astype(f32), adst0.astype(f32)], axis=1)
    xw0, ssd0 = _tc_pre(h0, ln0_g.reshape(1, D), ln0_b.reshape(1, D),
                        W0.astype(f32), a20)
    ue0 = _edge_pass(src, dst, ea, ssd0[:, 0], ssd0[:, 1], xw0, zero_u)

    a21 = jnp.stack([asrc1.astype(f32), adst1.astype(f32)], axis=1)
    xw1, ssd1 = _tc_mid(ue0, ln1_g.reshape(1, D), ln1_b.reshape(1, D),
                        W1.astype(f32), a21)
    ue1 = _edge_pass(src, dst, ea, ssd1[:, 0], ssd1[:, 1], xw1, zero_u)

    return _tc_fin(ue1, mW1.astype(f32), mb1.reshape(1, D // 2),
                   mW2.astype(f32), mb2.reshape(1, 2))
